# baseline (speedup 1.0000x reference)
import functools

import numpy as np
import jax
import jax.numpy as jnp
from jax import lax
from jax.experimental import pallas as pl
from jax.experimental.pallas import tpu as pltpu
from jax.experimental.pallas import tpu_sc as plsc

N = 10000
E = 320000
D = 128
DE = 16
H = 16
HD = 16
F = 256
SCALE = 1.0 / float(np.sqrt(HD))

NC = 2
NS = 16
NW = NC * NS
EPW = E // NW
CE = 16
NCHUNK = EPW // CE
NP = 10240
NPP = NP // 8
RPT = NP // NS
RPTD = NPP // NS


def _node_proj_body(x_ref, wq_ref, bq_ref, wk_ref, bk_ref, wv_ref, wtp_ref,
                    q_ref, k_ref, p_ref):
    x = x_ref[...]
    q_ref[...] = x @ wq_ref[...] + bq_ref[...]
    k_ref[...] = x @ wk_ref[...] + bk_ref[...]
    p_ref[...] = (x @ wv_ref[...]) @ wtp_ref[...]


def _node_proj(nf, wq, bq, wk, bk, wv, wtp0):
    blk = 1000
    grid = N // blk
    return pl.pallas_call(
        _node_proj_body,
        grid=(grid,),
        in_specs=[
            pl.BlockSpec((blk, D), lambda i: (i, 0)),
            pl.BlockSpec((D, F), lambda i: (0, 0)),
            pl.BlockSpec((1, F), lambda i: (0, 0)),
            pl.BlockSpec((D, F), lambda i: (0, 0)),
            pl.BlockSpec((1, F), lambda i: (0, 0)),
            pl.BlockSpec((D, D), lambda i: (0, 0)),
            pl.BlockSpec((D, D), lambda i: (0, 0)),
        ],
        out_specs=[
            pl.BlockSpec((blk, F), lambda i: (i, 0)),
            pl.BlockSpec((blk, F), lambda i: (i, 0)),
            pl.BlockSpec((blk, D), lambda i: (i, 0)),
        ],
        out_shape=[
            jax.ShapeDtypeStruct((N, F), jnp.float32),
            jax.ShapeDtypeStruct((N, F), jnp.float32),
            jax.ShapeDtypeStruct((N, D), jnp.float32),
        ],
    )(nf, wq, bq, wk, bk, wv, wtp0)


def _edge_proj_body(ea_ref, we_ref, be_ref, sh_ref, ef_ref, shx_ref):
    ef_ref[...] = ea_ref[...] @ we_ref[...] + be_ref[...]
    shx_ref[...] = jnp.broadcast_to(sh_ref[...], shx_ref.shape)


def _edge_proj(ea, we, be, sh):
    blk = 4000
    grid = E // blk
    return pl.pallas_call(
        _edge_proj_body,
        grid=(grid,),
        in_specs=[
            pl.BlockSpec((blk, DE), lambda i: (i, 0)),
            pl.BlockSpec((DE, F), lambda i: (0, 0)),
            pl.BlockSpec((1, F), lambda i: (0, 0)),
            pl.BlockSpec((blk, 1), lambda i: (i, 0)),
        ],
        out_specs=[
            pl.BlockSpec((blk, F), lambda i: (i, 0)),
            pl.BlockSpec((blk, H), lambda i: (i, 0)),
        ],
        out_shape=[
            jax.ShapeDtypeStruct((E, F), jnp.float32),
            jax.ShapeDtypeStruct((E, H), jnp.float32),
        ],
    )(ea, we, be, sh)


_GDN = lax.GatherDimensionNumbers(offset_dims=(), collapsed_slice_dims=(0,),
                                  start_index_map=(0,))


def _shuf(v, idx16):
    return lax.gather(v, idx16[:, None], _GDN, (1,),
                      mode=lax.GatherScatterMode.PROMISE_IN_BOUNDS)


def _splat_lane(v, lane):
    return _shuf(v, jnp.full((16,), lane, jnp.int32))


def _splat_sum(x):
    for b in (1, 2, 4, 8):
        x = x + _shuf(x, jnp.bitwise_xor(lax.iota(jnp.int32, 16), b))
    return x


def _rsqrt(x):
    xi = lax.bitcast_convert_type(x, jnp.int32)
    yi = jnp.int32(0x5F3759DF) - lax.shift_right_logical(xi, 1)
    y = lax.bitcast_convert_type(yi, jnp.float32)
    for _ in range(3):
        y = y * (1.5 - 0.5 * x * y * y)
    return y


_MESH = plsc.VectorSubcoreMesh(core_axis_name="c", subcore_axis_name="s")


@functools.partial(
    pl.kernel,
    out_type=[
        jax.ShapeDtypeStruct((E, H), jnp.float32),
        jax.ShapeDtypeStruct((NC, NPP, D), jnp.float32),
    ],
    mesh=_MESH,
    scratch_types=[
        pltpu.VMEM((CE,), jnp.int32),
        pltpu.VMEM((CE,), jnp.int32),
        pltpu.VMEM((CE, F), jnp.float32),
        pltpu.VMEM((CE, F), jnp.float32),
        pltpu.VMEM((CE, F), jnp.float32),
        pltpu.VMEM((CE, H), jnp.float32),
        pltpu.VMEM((CE, D), jnp.float32),
        pltpu.VMEM_SHARED((NPP, D), jnp.float32),
        pltpu.SemaphoreType.DMA,
        pltpu.SemaphoreType.DMA,
    ],
)
def _pass_a(qt_hbm, kt_hbm, eft_hbm, dst_hbm, src_hbm, zden_hbm,
            e_hbm, den_hbm,
            dstv, srcv, qd, ks, ef, ev, evw, den_sh, sem0, sem1):
    c = lax.axis_index("c")
    s = lax.axis_index("s")
    wid = c * NS + s
    rows = pl.ds(s * RPTD, RPTD)
    pltpu.sync_copy(zden_hbm.at[rows], den_sh.at[rows])
    plsc.subcore_barrier()

    def chunk_body(ci, carry):
        base = wid * EPW + ci * CE
        pltpu.sync_copy(dst_hbm.at[pl.ds(base, CE)], dstv)
        pltpu.sync_copy(src_hbm.at[pl.ds(base, CE)], srcv)
        cq = pltpu.async_copy(qt_hbm.at[dstv], qd, sem0)
        ck = pltpu.async_copy(kt_hbm.at[srcv], ks, sem1)
        pltpu.sync_copy(eft_hbm.at[pl.ds(base, CE)], ef)
        cq.wait()
        ck.wait()
        dvec = dstv[...]
        seg = jnp.bitwise_and(dvec, 7)
        for l in range(CE):
            acc = jnp.zeros((16,), jnp.float32)
            for hd in range(HD):
                sl = pl.ds(hd * 16, 16)
                acc = acc + qd[l, sl] * (ks[l, sl] + ef[l, sl])
            e_vec = jnp.exp(acc * SCALE)
            ev[l, :] = e_vec
            segl = _splat_lane(seg, l)
            for k in range(8):
                ind = jnp.minimum(jnp.abs(segl - k), 1)
                indf = (1 - ind).astype(jnp.float32)
                evw[l, pl.ds(k * 16, 16)] = e_vec * indf
        pltpu.sync_copy(ev, e_hbm.at[pl.ds(base, CE)])
        ireg = lax.shift_right_logical(dvec, 3)
        pltpu.sync_copy(evw, den_sh.at[ireg], add=True)
        return carry

    lax.fori_loop(0, NCHUNK, chunk_body, 0)
    plsc.subcore_barrier()
    pltpu.sync_copy(den_sh.at[rows], den_hbm.at[c, rows])


def _den_comb_body(d_ref, dc_ref):
    dc_ref[...] = d_ref[0] + d_ref[1]


def _den_comb(d):
    blk = 128
    grid = NPP // blk
    return pl.pallas_call(
        _den_comb_body,
        grid=(grid,),
        in_specs=[pl.BlockSpec((NC, blk, D), lambda i: (0, i, 0))],
        out_specs=pl.BlockSpec((blk, D), lambda i: (i, 0)),
        out_shape=jax.ShapeDtypeStruct((NPP, D), jnp.float32),
    )(d)


@functools.partial(
    pl.kernel,
    out_type=jax.ShapeDtypeStruct((NC, NP, D), jnp.float32),
    mesh=_MESH,
    scratch_types=[
        pltpu.VMEM((CE,), jnp.int32),
        pltpu.VMEM((CE,), jnp.int32),
        pltpu.VMEM((CE, H), jnp.float32),
        pltpu.VMEM((CE, D), jnp.float32),
        pltpu.VMEM((CE, D), jnp.float32),
        pltpu.VMEM((CE, H), jnp.float32),
        pltpu.VMEM((CE, D), jnp.float32),
        pltpu.VMEM((D,), jnp.float32),
        pltpu.VMEM((D,), jnp.float32),
        pltpu.VMEM((D,), jnp.float32),
        pltpu.VMEM_SHARED((NP, D), jnp.float32),
        pltpu.SemaphoreType.DMA,
        pltpu.SemaphoreType.DMA,
    ],
)
def _pass_b(e_hbm, denc_hbm, p_hbm, shx_hbm, dst_hbm, src_hbm,
            btp_hbm, gm_hbm, bm_hbm, zagg_hbm,
            agg_hbm,
            dstv, srcv, ev, dw, pv, shv, msg, btp_v, gm_v, bm_v,
            agg_sh, sem0, sem2):
    c = lax.axis_index("c")
    s = lax.axis_index("s")
    wid = c * NS + s
    rows = pl.ds(s * RPT, RPT)
    pltpu.sync_copy(zagg_hbm.at[rows], agg_sh.at[rows])
    pltpu.sync_copy(btp_hbm, btp_v)
    pltpu.sync_copy(gm_hbm, gm_v)
    pltpu.sync_copy(bm_hbm, bm_v)
    plsc.subcore_barrier()

    def chunk_body(ci, carry):
        base = wid * EPW + ci * CE
        pltpu.sync_copy(dst_hbm.at[pl.ds(base, CE)], dstv)
        pltpu.sync_copy(src_hbm.at[pl.ds(base, CE)], srcv)
        dvec = dstv[...]
        c0 = pltpu.async_copy(denc_hbm.at[lax.shift_right_logical(dvec, 3)],
                              dw, sem0)
        cp = pltpu.async_copy(p_hbm.at[srcv], pv, sem2)
        pltpu.sync_copy(e_hbm.at[pl.ds(base, CE)], ev)
        pltpu.sync_copy(shx_hbm.at[pl.ds(base, CE)], shv)
        c0.wait()
        cp.wait()
        seg = jnp.bitwise_and(dvec, 7)
        zero16 = jnp.zeros((16,), jnp.float32)
        for l in range(CE):
            segl = _splat_lane(seg, l)
            dd = zero16
            for k in range(8):
                ind = jnp.minimum(jnp.abs(segl - k), 1)
                indf = (1 - ind).astype(jnp.float32)
                dd = dd + dw[l, pl.ds(k * 16, 16)] * indf
            y = ev[l, :] / dd
            aw = _splat_sum(y) * (1.0 / H)
            shl = shv[l, :]
            vs = []
            tot = zero16
            for j in range(D // 16):
                sl = pl.ds(j * 16, 16)
                m = pv[l, sl] * shl + btp_v[sl]
                vs.append(m)
                tot = tot + m
            mu = _splat_sum(tot) * (1.0 / D)
            sq = zero16
            for j in range(D // 16):
                dm = vs[j] - mu
                vs[j] = dm
                sq = sq + dm * dm
            rinv = _rsqrt(_splat_sum(sq) * (1.0 / D) + 1e-5)
            for j in range(D // 16):
                sl = pl.ds(j * 16, 16)
                z = vs[j] * rinv * gm_v[sl] + bm_v[sl]
                zs = z / (1.0 + jnp.exp(-z))
                msg[l, sl] = zs * aw
        pltpu.sync_copy(msg, agg_sh.at[dvec], add=True)
        return carry

    lax.fori_loop(0, NCHUNK, chunk_body, 0)
    plsc.subcore_barrier()
    pltpu.sync_copy(agg_sh.at[rows], agg_hbm.at[c, rows])


def _ln(x, g, b):
    mu = jnp.mean(x, axis=-1, keepdims=True)
    var = jnp.mean((x - mu) ** 2, axis=-1, keepdims=True)
    return (x - mu) * lax.rsqrt(var + 1e-5) * g + b


def _post_body(nf_ref, a_ref, wout_ref, g1_ref, b1_ref,
               wf1_ref, wf2_ref, g2_ref, b2_ref, out_ref):
    agg = a_ref[0] + a_ref[1]
    h1 = nf_ref[...] + agg @ wout_ref[...]
    h1 = _ln(h1, g1_ref[...], b1_ref[...])
    f = h1 @ wf1_ref[...]
    f = (f / (1.0 + jnp.exp(-f))) @ wf2_ref[...]
    out_ref[...] = _ln(h1 + f, g2_ref[...], b2_ref[...])


def _post(nf, a, wout, g1, b1, wf1, wf2, g2, b2):
    blk = 1000
    grid = N // blk
    row = lambda i: (i, 0)
    full = lambda i: (0, 0)
    return pl.pallas_call(
        _post_body,
        grid=(grid,),
        in_specs=[
            pl.BlockSpec((blk, D), row),
            pl.BlockSpec((NC, blk, D), lambda i: (0, i, 0)),
            pl.BlockSpec((D, D), full),
            pl.BlockSpec((1, D), full),
            pl.BlockSpec((1, D), full),
            pl.BlockSpec((D, D), full),
            pl.BlockSpec((D, D), full),
            pl.BlockSpec((1, D), full),
            pl.BlockSpec((1, D), full),
        ],
        out_specs=pl.BlockSpec((blk, D), row),
        out_shape=jax.ShapeDtypeStruct((N, D), jnp.float32),
    )(nf, a, wout, g1, b1, wf1, wf2, g2, b2)


_PERM = np.arange(F).reshape(H, HD).T.reshape(-1)


def kernel(node_features, edge_index, edge_attr, edge_sh, batch,
           Wq, bq, Wk, bk, Wv, Wedge, bedge, Wtp, btp, gm, bm,
           Wout, g1, b1, Wffn1, Wffn2, g2, b2):
    perm = jnp.asarray(_PERM)
    wq = Wq[:, perm]
    bqp = bq[perm].reshape(1, F)
    wk = Wk[:, perm]
    bkp = bk[perm].reshape(1, F)
    we = Wedge[:, perm]
    bep = bedge[perm].reshape(1, F)
    wtp0 = Wtp[:, 0, :]
    src = edge_index[0]
    dst = edge_index[1]
    zden = jnp.zeros((NPP, D), jnp.float32)
    zagg = jnp.zeros((NP, D), jnp.float32)

    qt, kt, p = _node_proj(node_features, wq, bqp, wk, bkp, Wv, wtp0)
    eft, shx = _edge_proj(edge_attr, we, bep, edge_sh)
    e, den = _pass_a(qt, kt, eft, dst, src, zden)
    denc = _den_comb(den)
    agg = _pass_b(e, denc, p, shx, dst, src, btp, gm, bm, zagg)
    return _post(node_features, agg, Wout,
                 g1.reshape(1, D), b1.reshape(1, D),
                 Wffn1, Wffn2, g2.reshape(1, D), b2.reshape(1, D))


# speedup vs baseline: 1.1952x; 1.1952x over previous
import functools

import numpy as np
import jax
import jax.numpy as jnp
from jax import lax
from jax.experimental import pallas as pl
from jax.experimental.pallas import tpu as pltpu
from jax.experimental.pallas import tpu_sc as plsc

N = 10000
E = 320000
D = 128
DE = 16
H = 16
HD = 16
F = 256
SCALE = 1.0 / float(np.sqrt(HD))

NC = 2
NS = 16
NW = NC * NS
EPW = E // NW
CE = 16
NCHUNK = EPW // CE
NP = 10240
NPP = NP // 8
RPT = NP // NS
RPTD = NPP // NS


def _node_proj_body(x_ref, wq_ref, bq_ref, wk_ref, bk_ref, wv_ref, wtp_ref,
                    q_ref, k_ref, p_ref):
    x = x_ref[...]
    q_ref[...] = x @ wq_ref[...] + bq_ref[...]
    k_ref[...] = x @ wk_ref[...] + bk_ref[...]
    p_ref[...] = (x @ wv_ref[...]) @ wtp_ref[...]


def _node_proj(nf, wq, bq, wk, bk, wv, wtp0):
    blk = 1000
    grid = N // blk
    return pl.pallas_call(
        _node_proj_body,
        grid=(grid,),
        in_specs=[
            pl.BlockSpec((blk, D), lambda i: (i, 0)),
            pl.BlockSpec((D, F), lambda i: (0, 0)),
            pl.BlockSpec((1, F), lambda i: (0, 0)),
            pl.BlockSpec((D, F), lambda i: (0, 0)),
            pl.BlockSpec((1, F), lambda i: (0, 0)),
            pl.BlockSpec((D, D), lambda i: (0, 0)),
            pl.BlockSpec((D, D), lambda i: (0, 0)),
        ],
        out_specs=[
            pl.BlockSpec((blk, F), lambda i: (i, 0)),
            pl.BlockSpec((blk, F), lambda i: (i, 0)),
            pl.BlockSpec((blk, D), lambda i: (i, 0)),
        ],
        out_shape=[
            jax.ShapeDtypeStruct((N, F), jnp.float32),
            jax.ShapeDtypeStruct((N, F), jnp.float32),
            jax.ShapeDtypeStruct((N, D), jnp.float32),
        ],
    )(nf, wq, bq, wk, bk, wv, wtp0)


def _edge_proj_body(ea_ref, we_ref, be_ref, sh_ref, ef_ref, shx_ref):
    ef_ref[...] = ea_ref[...] @ we_ref[...] + be_ref[...]
    shx_ref[...] = jnp.broadcast_to(sh_ref[...], shx_ref.shape)


def _edge_proj(ea, we, be, sh):
    blk = 4000
    grid = E // blk
    return pl.pallas_call(
        _edge_proj_body,
        grid=(grid,),
        in_specs=[
            pl.BlockSpec((blk, DE), lambda i: (i, 0)),
            pl.BlockSpec((DE, F), lambda i: (0, 0)),
            pl.BlockSpec((1, F), lambda i: (0, 0)),
            pl.BlockSpec((blk, 1), lambda i: (i, 0)),
        ],
        out_specs=[
            pl.BlockSpec((blk, F), lambda i: (i, 0)),
            pl.BlockSpec((blk, H), lambda i: (i, 0)),
        ],
        out_shape=[
            jax.ShapeDtypeStruct((E, F), jnp.float32),
            jax.ShapeDtypeStruct((E, H), jnp.float32),
        ],
    )(ea, we, be, sh)


_GDN = lax.GatherDimensionNumbers(offset_dims=(), collapsed_slice_dims=(0,),
                                  start_index_map=(0,))


def _shuf(v, idx16):
    return lax.gather(v, idx16[:, None], _GDN, (1,),
                      mode=lax.GatherScatterMode.PROMISE_IN_BOUNDS)


def _splat_lane(v, lane):
    return _shuf(v, jnp.full((16,), lane, jnp.int32))


def _splat_sum(x):
    for b in (1, 2, 4, 8):
        x = x + _shuf(x, jnp.bitwise_xor(lax.iota(jnp.int32, 16), b))
    return x


def _rsqrt(x):
    xi = lax.bitcast_convert_type(x, jnp.int32)
    yi = jnp.int32(0x5F3759DF) - lax.shift_right_logical(xi, 1)
    y = lax.bitcast_convert_type(yi, jnp.float32)
    for _ in range(3):
        y = y * (1.5 - 0.5 * x * y * y)
    return y


_MESH = plsc.VectorSubcoreMesh(core_axis_name="c", subcore_axis_name="s")


@functools.partial(
    pl.kernel,
    out_type=[
        jax.ShapeDtypeStruct((E, H), jnp.float32),
        jax.ShapeDtypeStruct((NC, NPP, D), jnp.float32),
    ],
    mesh=_MESH,
    scratch_types=[
        pltpu.VMEM((EPW,), jnp.int32),
        pltpu.VMEM((EPW,), jnp.int32),
        pltpu.VMEM((2, CE, F), jnp.float32),
        pltpu.VMEM((2, CE, F), jnp.float32),
        pltpu.VMEM((2, CE, F), jnp.float32),
        pltpu.VMEM((CE, H), jnp.float32),
        pltpu.VMEM((CE, D), jnp.float32),
        pltpu.VMEM_SHARED((NPP, D), jnp.float32),
        pltpu.SemaphoreType.DMA,
        pltpu.SemaphoreType.DMA,
    ],
)
def _pass_a(qt_hbm, kt_hbm, eft_hbm, dst_hbm, src_hbm, zden_hbm,
            e_hbm, den_hbm,
            dsta, srca, qd, ks, ef, ev, evw, den_sh, semA, semB):
    c = lax.axis_index("c")
    s = lax.axis_index("s")
    wid = c * NS + s
    rows = pl.ds(s * RPTD, RPTD)
    pltpu.sync_copy(zden_hbm.at[rows], den_sh.at[rows])
    pltpu.sync_copy(dst_hbm.at[pl.ds(wid * EPW, EPW)], dsta)
    pltpu.sync_copy(src_hbm.at[pl.ds(wid * EPW, EPW)], srca)
    plsc.subcore_barrier()
    sems = (semA, semB)

    def issue(ci, b):
        off = ci * CE
        dreg = dsta[pl.ds(off, CE)]
        sreg = srca[pl.ds(off, CE)]
        pltpu.async_copy(qt_hbm.at[dreg], qd.at[b], sems[b])
        pltpu.async_copy(kt_hbm.at[sreg], ks.at[b], sems[b])
        pltpu.async_copy(eft_hbm.at[pl.ds(wid * EPW + off, CE)], ef.at[b],
                         sems[b])

    def drain(b):
        pltpu.make_async_copy(qt_hbm.at[pl.ds(0, CE)], qd.at[b],
                              sems[b]).wait()
        pltpu.make_async_copy(kt_hbm.at[pl.ds(0, CE)], ks.at[b],
                              sems[b]).wait()
        pltpu.make_async_copy(eft_hbm.at[pl.ds(0, CE)], ef.at[b],
                              sems[b]).wait()

    def compute(ci, b):
        off = ci * CE
        dvec = dsta[pl.ds(off, CE)]
        seg = jnp.bitwise_and(dvec, 7)
        for l in range(CE):
            acc = jnp.zeros((16,), jnp.float32)
            for hd in range(HD):
                sl = pl.ds(hd * 16, 16)
                acc = acc + qd[b, l, sl] * (ks[b, l, sl] + ef[b, l, sl])
            e_vec = jnp.exp(acc * SCALE)
            ev[l, :] = e_vec
            segl = _splat_lane(seg, l)
            for k in range(8):
                ind = jnp.minimum(jnp.abs(segl - k), 1)
                indf = (1 - ind).astype(jnp.float32)
                evw[l, pl.ds(k * 16, 16)] = e_vec * indf
        pltpu.sync_copy(ev, e_hbm.at[pl.ds(wid * EPW + off, CE)])
        ireg = lax.shift_right_logical(dvec, 3)
        pltpu.sync_copy(evw, den_sh.at[ireg], add=True)

    issue(0, 0)

    def pair_body(t, carry):
        c0 = 2 * t
        issue(c0 + 1, 1)
        drain(0)
        compute(c0, 0)
        issue(c0 + 2, 0)
        drain(1)
        compute(c0 + 1, 1)
        return carry

    lax.fori_loop(0, (NCHUNK - 1) // 2, pair_body, 0)
    drain(0)
    compute(NCHUNK - 1, 0)
    plsc.subcore_barrier()
    pltpu.sync_copy(den_sh.at[rows], den_hbm.at[c, rows])


def _den_comb_body(d_ref, dc_ref):
    dc_ref[...] = d_ref[0] + d_ref[1]


def _den_comb(d):
    blk = 128
    grid = NPP // blk
    return pl.pallas_call(
        _den_comb_body,
        grid=(grid,),
        in_specs=[pl.BlockSpec((NC, blk, D), lambda i: (0, i, 0))],
        out_specs=pl.BlockSpec((blk, D), lambda i: (i, 0)),
        out_shape=jax.ShapeDtypeStruct((NPP, D), jnp.float32),
    )(d)


@functools.partial(
    pl.kernel,
    out_type=jax.ShapeDtypeStruct((NC, NP, D), jnp.float32),
    mesh=_MESH,
    scratch_types=[
        pltpu.VMEM((CE,), jnp.int32),
        pltpu.VMEM((CE,), jnp.int32),
        pltpu.VMEM((CE, H), jnp.float32),
        pltpu.VMEM((CE, D), jnp.float32),
        pltpu.VMEM((CE, D), jnp.float32),
        pltpu.VMEM((CE, H), jnp.float32),
        pltpu.VMEM((CE, D), jnp.float32),
        pltpu.VMEM((D,), jnp.float32),
        pltpu.VMEM((D,), jnp.float32),
        pltpu.VMEM((D,), jnp.float32),
        pltpu.VMEM_SHARED((NP, D), jnp.float32),
        pltpu.SemaphoreType.DMA,
        pltpu.SemaphoreType.DMA,
    ],
)
def _pass_b(e_hbm, denc_hbm, p_hbm, shx_hbm, dst_hbm, src_hbm,
            btp_hbm, gm_hbm, bm_hbm, zagg_hbm,
            agg_hbm,
            dstv, srcv, ev, dw, pv, shv, msg, btp_v, gm_v, bm_v,
            agg_sh, sem0, sem2):
    c = lax.axis_index("c")
    s = lax.axis_index("s")
    wid = c * NS + s
    rows = pl.ds(s * RPT, RPT)
    pltpu.sync_copy(zagg_hbm.at[rows], agg_sh.at[rows])
    pltpu.sync_copy(btp_hbm, btp_v)
    pltpu.sync_copy(gm_hbm, gm_v)
    pltpu.sync_copy(bm_hbm, bm_v)
    plsc.subcore_barrier()

    def chunk_body(ci, carry):
        base = wid * EPW + ci * CE
        pltpu.sync_copy(dst_hbm.at[pl.ds(base, CE)], dstv)
        pltpu.sync_copy(src_hbm.at[pl.ds(base, CE)], srcv)
        dvec = dstv[...]
        c0 = pltpu.async_copy(denc_hbm.at[lax.shift_right_logical(dvec, 3)],
                              dw, sem0)
        cp = pltpu.async_copy(p_hbm.at[srcv], pv, sem2)
        pltpu.sync_copy(e_hbm.at[pl.ds(base, CE)], ev)
        pltpu.sync_copy(shx_hbm.at[pl.ds(base, CE)], shv)
        c0.wait()
        cp.wait()
        seg = jnp.bitwise_and(dvec, 7)
        zero16 = jnp.zeros((16,), jnp.float32)
        for l in range(CE):
            segl = _splat_lane(seg, l)
            dd = zero16
            for k in range(8):
                ind = jnp.minimum(jnp.abs(segl - k), 1)
                indf = (1 - ind).astype(jnp.float32)
                dd = dd + dw[l, pl.ds(k * 16, 16)] * indf
            y = ev[l, :] / dd
            aw = _splat_sum(y) * (1.0 / H)
            shl = shv[l, :]
            vs = []
            tot = zero16
            for j in range(D // 16):
                sl = pl.ds(j * 16, 16)
                m = pv[l, sl] * shl + btp_v[sl]
                vs.append(m)
                tot = tot + m
            mu = _splat_sum(tot) * (1.0 / D)
            sq = zero16
            for j in range(D // 16):
                dm = vs[j] - mu
                vs[j] = dm
                sq = sq + dm * dm
            rinv = _rsqrt(_splat_sum(sq) * (1.0 / D) + 1e-5)
            for j in range(D // 16):
                sl = pl.ds(j * 16, 16)
                z = vs[j] * rinv * gm_v[sl] + bm_v[sl]
                zs = z / (1.0 + jnp.exp(-z))
                msg[l, sl] = zs * aw
        pltpu.sync_copy(msg, agg_sh.at[dvec], add=True)
        return carry

    lax.fori_loop(0, NCHUNK, chunk_body, 0)
    plsc.subcore_barrier()
    pltpu.sync_copy(agg_sh.at[rows], agg_hbm.at[c, rows])


def _ln(x, g, b):
    mu = jnp.mean(x, axis=-1, keepdims=True)
    var = jnp.mean((x - mu) ** 2, axis=-1, keepdims=True)
    return (x - mu) * lax.rsqrt(var + 1e-5) * g + b


def _post_body(nf_ref, a_ref, wout_ref, g1_ref, b1_ref,
               wf1_ref, wf2_ref, g2_ref, b2_ref, out_ref):
    agg = a_ref[0] + a_ref[1]
    h1 = nf_ref[...] + agg @ wout_ref[...]
    h1 = _ln(h1, g1_ref[...], b1_ref[...])
    f = h1 @ wf1_ref[...]
    f = (f / (1.0 + jnp.exp(-f))) @ wf2_ref[...]
    out_ref[...] = _ln(h1 + f, g2_ref[...], b2_ref[...])


def _post(nf, a, wout, g1, b1, wf1, wf2, g2, b2):
    blk = 1000
    grid = N // blk
    row = lambda i: (i, 0)
    full = lambda i: (0, 0)
    return pl.pallas_call(
        _post_body,
        grid=(grid,),
        in_specs=[
            pl.BlockSpec((blk, D), row),
            pl.BlockSpec((NC, blk, D), lambda i: (0, i, 0)),
            pl.BlockSpec((D, D), full),
            pl.BlockSpec((1, D), full),
            pl.BlockSpec((1, D), full),
            pl.BlockSpec((D, D), full),
            pl.BlockSpec((D, D), full),
            pl.BlockSpec((1, D), full),
            pl.BlockSpec((1, D), full),
        ],
        out_specs=pl.BlockSpec((blk, D), row),
        out_shape=jax.ShapeDtypeStruct((N, D), jnp.float32),
    )(nf, a, wout, g1, b1, wf1, wf2, g2, b2)


_PERM = np.arange(F).reshape(H, HD).T.reshape(-1)


def kernel(node_features, edge_index, edge_attr, edge_sh, batch,
           Wq, bq, Wk, bk, Wv, Wedge, bedge, Wtp, btp, gm, bm,
           Wout, g1, b1, Wffn1, Wffn2, g2, b2):
    perm = jnp.asarray(_PERM)
    wq = Wq[:, perm]
    bqp = bq[perm].reshape(1, F)
    wk = Wk[:, perm]
    bkp = bk[perm].reshape(1, F)
    we = Wedge[:, perm]
    bep = bedge[perm].reshape(1, F)
    wtp0 = Wtp[:, 0, :]
    src = edge_index[0]
    dst = edge_index[1]
    zden = jnp.zeros((NPP, D), jnp.float32)
    zagg = jnp.zeros((NP, D), jnp.float32)

    qt, kt, p = _node_proj(node_features, wq, bqp, wk, bkp, Wv, wtp0)
    eft, shx = _edge_proj(edge_attr, we, bep, edge_sh)
    e, den = _pass_a(qt, kt, eft, dst, src, zden)
    denc = _den_comb(den)
    agg = _pass_b(e, denc, p, shx, dst, src, btp, gm, bm, zagg)
    return _post(node_features, agg, Wout,
                 g1.reshape(1, D), b1.reshape(1, D),
                 Wffn1, Wffn2, g2.reshape(1, D), b2.reshape(1, D))


# speedup vs baseline: 1.3245x; 1.1081x over previous
import functools

import numpy as np
import jax
import jax.numpy as jnp
from jax import lax
from jax.experimental import pallas as pl
from jax.experimental.pallas import tpu as pltpu
from jax.experimental.pallas import tpu_sc as plsc

N = 10000
E = 320000
D = 128
DE = 16
H = 16
HD = 16
F = 256
SCALE = 1.0 / float(np.sqrt(HD))

NC = 2
NS = 16
NW = NC * NS
EPW = E // NW
CE = 16
NCHUNK = EPW // CE
NP = 10240
NPP = NP // 8
RPT = NP // NS
RPTD = NPP // NS


def _node_proj_body(x_ref, wq_ref, bq_ref, wk_ref, bk_ref, wv_ref, wtp_ref,
                    q_ref, k_ref, p_ref):
    x = x_ref[...]
    q_ref[...] = x @ wq_ref[...] + bq_ref[...]
    k_ref[...] = x @ wk_ref[...] + bk_ref[...]
    p_ref[...] = (x @ wv_ref[...]) @ wtp_ref[...]


def _node_proj(nf, wq, bq, wk, bk, wv, wtp0):
    blk = 1000
    grid = N // blk
    return pl.pallas_call(
        _node_proj_body,
        grid=(grid,),
        in_specs=[
            pl.BlockSpec((blk, D), lambda i: (i, 0)),
            pl.BlockSpec((D, F), lambda i: (0, 0)),
            pl.BlockSpec((1, F), lambda i: (0, 0)),
            pl.BlockSpec((D, F), lambda i: (0, 0)),
            pl.BlockSpec((1, F), lambda i: (0, 0)),
            pl.BlockSpec((D, D), lambda i: (0, 0)),
            pl.BlockSpec((D, D), lambda i: (0, 0)),
        ],
        out_specs=[
            pl.BlockSpec((blk, F), lambda i: (i, 0)),
            pl.BlockSpec((blk, F), lambda i: (i, 0)),
            pl.BlockSpec((blk, D), lambda i: (i, 0)),
        ],
        out_shape=[
            jax.ShapeDtypeStruct((N, F), jnp.float32),
            jax.ShapeDtypeStruct((N, F), jnp.float32),
            jax.ShapeDtypeStruct((N, D), jnp.float32),
        ],
    )(nf, wq, bq, wk, bk, wv, wtp0)


def _edge_proj_body(ea_ref, we_ref, be_ref, sh_ref, ef_ref, shx_ref):
    ef_ref[...] = ea_ref[...] @ we_ref[...] + be_ref[...]
    shx_ref[...] = jnp.broadcast_to(sh_ref[...], shx_ref.shape)


def _edge_proj(ea, we, be, sh):
    blk = 4000
    grid = E // blk
    return pl.pallas_call(
        _edge_proj_body,
        grid=(grid,),
        in_specs=[
            pl.BlockSpec((blk, DE), lambda i: (i, 0)),
            pl.BlockSpec((DE, F), lambda i: (0, 0)),
            pl.BlockSpec((1, F), lambda i: (0, 0)),
            pl.BlockSpec((blk, 1), lambda i: (i, 0)),
        ],
        out_specs=[
            pl.BlockSpec((blk, F), lambda i: (i, 0)),
            pl.BlockSpec((blk, H), lambda i: (i, 0)),
        ],
        out_shape=[
            jax.ShapeDtypeStruct((E, F), jnp.float32),
            jax.ShapeDtypeStruct((E, H), jnp.float32),
        ],
    )(ea, we, be, sh)


_GDN = lax.GatherDimensionNumbers(offset_dims=(), collapsed_slice_dims=(0,),
                                  start_index_map=(0,))


def _shuf(v, idx16):
    return lax.gather(v, idx16[:, None], _GDN, (1,),
                      mode=lax.GatherScatterMode.PROMISE_IN_BOUNDS)


def _splat_lane(v, lane):
    return _shuf(v, jnp.full((16,), lane, jnp.int32))


def _splat_sum(x):
    for b in (1, 2, 4, 8):
        x = x + _shuf(x, jnp.bitwise_xor(lax.iota(jnp.int32, 16), b))
    return x


def _rsqrt(x):
    xi = lax.bitcast_convert_type(x, jnp.int32)
    yi = jnp.int32(0x5F3759DF) - lax.shift_right_logical(xi, 1)
    y = lax.bitcast_convert_type(yi, jnp.float32)
    for _ in range(3):
        y = y * (1.5 - 0.5 * x * y * y)
    return y


_MESH = plsc.VectorSubcoreMesh(core_axis_name="c", subcore_axis_name="s")


@functools.partial(
    pl.kernel,
    out_type=[
        jax.ShapeDtypeStruct((E, H), jnp.float32),
        jax.ShapeDtypeStruct((NC, NPP, D), jnp.float32),
    ],
    mesh=_MESH,
    scratch_types=[
        pltpu.VMEM((EPW,), jnp.int32),
        pltpu.VMEM((EPW,), jnp.int32),
        pltpu.VMEM((2, CE, F), jnp.float32),
        pltpu.VMEM((2, CE, F), jnp.float32),
        pltpu.VMEM((2, CE, F), jnp.float32),
        pltpu.VMEM((CE, H), jnp.float32),
        pltpu.VMEM((CE, D), jnp.float32),
        pltpu.VMEM_SHARED((NPP, D), jnp.float32),
        pltpu.SemaphoreType.DMA,
        pltpu.SemaphoreType.DMA,
    ],
)
def _pass_a(qt_hbm, kt_hbm, eft_hbm, dst_hbm, src_hbm, zden_hbm,
            e_hbm, den_hbm,
            dsta, srca, qd, ks, ef, ev, evw, den_sh, semA, semB):
    c = lax.axis_index("c")
    s = lax.axis_index("s")
    wid = c * NS + s
    rows = pl.ds(s * RPTD, RPTD)
    pltpu.sync_copy(zden_hbm.at[rows], den_sh.at[rows])
    pltpu.sync_copy(dst_hbm.at[pl.ds(wid * EPW, EPW)], dsta)
    pltpu.sync_copy(src_hbm.at[pl.ds(wid * EPW, EPW)], srca)
    plsc.subcore_barrier()
    sems = (semA, semB)

    def issue(ci, b):
        off = ci * CE
        dreg = dsta[pl.ds(off, CE)]
        sreg = srca[pl.ds(off, CE)]
        pltpu.async_copy(qt_hbm.at[dreg], qd.at[b], sems[b])
        pltpu.async_copy(kt_hbm.at[sreg], ks.at[b], sems[b])
        pltpu.async_copy(eft_hbm.at[pl.ds(wid * EPW + off, CE)], ef.at[b],
                         sems[b])

    def drain(b):
        pltpu.make_async_copy(qt_hbm.at[pl.ds(0, CE)], qd.at[b],
                              sems[b]).wait()
        pltpu.make_async_copy(kt_hbm.at[pl.ds(0, CE)], ks.at[b],
                              sems[b]).wait()
        pltpu.make_async_copy(eft_hbm.at[pl.ds(0, CE)], ef.at[b],
                              sems[b]).wait()

    def compute(ci, b):
        off = ci * CE
        dvec = dsta[pl.ds(off, CE)]
        seg = jnp.bitwise_and(dvec, 7)
        for l in range(CE):
            acc = jnp.zeros((16,), jnp.float32)
            for hd in range(HD):
                sl = pl.ds(hd * 16, 16)
                acc = acc + qd[b, l, sl] * (ks[b, l, sl] + ef[b, l, sl])
            e_vec = jnp.exp(acc * SCALE)
            ev[l, :] = e_vec
            segl = _splat_lane(seg, l)
            for k in range(8):
                ind = jnp.minimum(jnp.abs(segl - k), 1)
                indf = (1 - ind).astype(jnp.float32)
                evw[l, pl.ds(k * 16, 16)] = e_vec * indf
        pltpu.sync_copy(ev, e_hbm.at[pl.ds(wid * EPW + off, CE)])
        ireg = lax.shift_right_logical(dvec, 3)
        pltpu.sync_copy(evw, den_sh.at[ireg], add=True)

    issue(0, 0)

    def pair_body(t, carry):
        c0 = 2 * t
        issue(c0 + 1, 1)
        drain(0)
        compute(c0, 0)
        issue(c0 + 2, 0)
        drain(1)
        compute(c0 + 1, 1)
        return carry

    lax.fori_loop(0, (NCHUNK - 1) // 2, pair_body, 0)
    drain(0)
    compute(NCHUNK - 1, 0)
    plsc.subcore_barrier()
    pltpu.sync_copy(den_sh.at[rows], den_hbm.at[c, rows])


def _den_comb_body(d_ref, dc_ref):
    dc_ref[...] = d_ref[0] + d_ref[1]


def _den_comb(d):
    blk = 128
    grid = NPP // blk
    return pl.pallas_call(
        _den_comb_body,
        grid=(grid,),
        in_specs=[pl.BlockSpec((NC, blk, D), lambda i: (0, i, 0))],
        out_specs=pl.BlockSpec((blk, D), lambda i: (i, 0)),
        out_shape=jax.ShapeDtypeStruct((NPP, D), jnp.float32),
    )(d)


@functools.partial(
    pl.kernel,
    out_type=jax.ShapeDtypeStruct((NC, NP, D), jnp.float32),
    mesh=_MESH,
    scratch_types=[
        pltpu.VMEM((2, CE), jnp.int32),
        pltpu.VMEM((2, CE), jnp.int32),
        pltpu.VMEM((2, CE, H), jnp.float32),
        pltpu.VMEM((2, CE, D), jnp.float32),
        pltpu.VMEM((2, CE, D), jnp.float32),
        pltpu.VMEM((2, CE, H), jnp.float32),
        pltpu.VMEM((CE, D), jnp.float32),
        pltpu.VMEM((D,), jnp.float32),
        pltpu.VMEM((D,), jnp.float32),
        pltpu.VMEM((D,), jnp.float32),
        pltpu.VMEM_SHARED((NP, D), jnp.float32),
        pltpu.SemaphoreType.DMA,
        pltpu.SemaphoreType.DMA,
    ],
)
def _pass_b(e_hbm, denc_hbm, p_hbm, shx_hbm, dst_hbm, src_hbm,
            btp_hbm, gm_hbm, bm_hbm, zagg_hbm,
            agg_hbm,
            dstb, srcb, ev, dw, pv, shv, msg, btp_v, gm_v, bm_v,
            agg_sh, semA, semB):
    c = lax.axis_index("c")
    s = lax.axis_index("s")
    wid = c * NS + s
    rows = pl.ds(s * RPT, RPT)
    pltpu.sync_copy(zagg_hbm.at[rows], agg_sh.at[rows])
    pltpu.sync_copy(btp_hbm, btp_v)
    pltpu.sync_copy(gm_hbm, gm_v)
    pltpu.sync_copy(bm_hbm, bm_v)
    plsc.subcore_barrier()
    sems = (semA, semB)

    def issue(ci, b):
        base = wid * EPW + ci * CE
        pltpu.sync_copy(dst_hbm.at[pl.ds(base, CE)], dstb.at[b])
        pltpu.sync_copy(src_hbm.at[pl.ds(base, CE)], srcb.at[b])
        dvec = dstb[b, :]
        sreg = srcb[b, :]
        pltpu.async_copy(denc_hbm.at[lax.shift_right_logical(dvec, 3)],
                         dw.at[b], sems[b])
        pltpu.async_copy(p_hbm.at[sreg], pv.at[b], sems[b])
        pltpu.async_copy(e_hbm.at[pl.ds(base, CE)], ev.at[b], sems[b])
        pltpu.async_copy(shx_hbm.at[pl.ds(base, CE)], shv.at[b], sems[b])

    def drain(b):
        pltpu.make_async_copy(denc_hbm.at[pl.ds(0, CE)], dw.at[b],
                              sems[b]).wait()
        pltpu.make_async_copy(p_hbm.at[pl.ds(0, CE)], pv.at[b],
                              sems[b]).wait()
        pltpu.make_async_copy(e_hbm.at[pl.ds(0, CE)], ev.at[b],
                              sems[b]).wait()
        pltpu.make_async_copy(shx_hbm.at[pl.ds(0, CE)], shv.at[b],
                              sems[b]).wait()

    def compute(ci, b):
        dvec = dstb[b, :]
        seg = jnp.bitwise_and(dvec, 7)
        zero16 = jnp.zeros((16,), jnp.float32)
        for l in range(CE):
            segl = _splat_lane(seg, l)
            dd = zero16
            for k in range(8):
                ind = jnp.minimum(jnp.abs(segl - k), 1)
                indf = (1 - ind).astype(jnp.float32)
                dd = dd + dw[b, l, pl.ds(k * 16, 16)] * indf
            y = ev[b, l, :] / dd
            aw = _splat_sum(y) * (1.0 / H)
            shl = shv[b, l, :]
            vs = []
            tot = zero16
            for j in range(D // 16):
                sl = pl.ds(j * 16, 16)
                m = pv[b, l, sl] * shl + btp_v[sl]
                vs.append(m)
                tot = tot + m
            mu = _splat_sum(tot) * (1.0 / D)
            sq = zero16
            for j in range(D // 16):
                dm = vs[j] - mu
                vs[j] = dm
                sq = sq + dm * dm
            rinv = _rsqrt(_splat_sum(sq) * (1.0 / D) + 1e-5)
            for j in range(D // 16):
                sl = pl.ds(j * 16, 16)
                z = vs[j] * rinv * gm_v[sl] + bm_v[sl]
                zs = z / (1.0 + jnp.exp(-z))
                msg[l, sl] = zs * aw
        pltpu.sync_copy(msg, agg_sh.at[dvec], add=True)

    issue(0, 0)

    def pair_body(t, carry):
        c0 = 2 * t
        issue(c0 + 1, 1)
        drain(0)
        compute(c0, 0)
        issue(c0 + 2, 0)
        drain(1)
        compute(c0 + 1, 1)
        return carry

    lax.fori_loop(0, (NCHUNK - 1) // 2, pair_body, 0)
    drain(0)
    compute(NCHUNK - 1, 0)
    plsc.subcore_barrier()
    pltpu.sync_copy(agg_sh.at[rows], agg_hbm.at[c, rows])


def _ln(x, g, b):
    mu = jnp.mean(x, axis=-1, keepdims=True)
    var = jnp.mean((x - mu) ** 2, axis=-1, keepdims=True)
    return (x - mu) * lax.rsqrt(var + 1e-5) * g + b


def _post_body(nf_ref, a_ref, wout_ref, g1_ref, b1_ref,
               wf1_ref, wf2_ref, g2_ref, b2_ref, out_ref):
    agg = a_ref[0] + a_ref[1]
    h1 = nf_ref[...] + agg @ wout_ref[...]
    h1 = _ln(h1, g1_ref[...], b1_ref[...])
    f = h1 @ wf1_ref[...]
    f = (f / (1.0 + jnp.exp(-f))) @ wf2_ref[...]
    out_ref[...] = _ln(h1 + f, g2_ref[...], b2_ref[...])


def _post(nf, a, wout, g1, b1, wf1, wf2, g2, b2):
    blk = 1000
    grid = N // blk
    row = lambda i: (i, 0)
    full = lambda i: (0, 0)
    return pl.pallas_call(
        _post_body,
        grid=(grid,),
        in_specs=[
            pl.BlockSpec((blk, D), row),
            pl.BlockSpec((NC, blk, D), lambda i: (0, i, 0)),
            pl.BlockSpec((D, D), full),
            pl.BlockSpec((1, D), full),
            pl.BlockSpec((1, D), full),
            pl.BlockSpec((D, D), full),
            pl.BlockSpec((D, D), full),
            pl.BlockSpec((1, D), full),
            pl.BlockSpec((1, D), full),
        ],
        out_specs=pl.BlockSpec((blk, D), row),
        out_shape=jax.ShapeDtypeStruct((N, D), jnp.float32),
    )(nf, a, wout, g1, b1, wf1, wf2, g2, b2)


_PERM = np.arange(F).reshape(H, HD).T.reshape(-1)


def kernel(node_features, edge_index, edge_attr, edge_sh, batch,
           Wq, bq, Wk, bk, Wv, Wedge, bedge, Wtp, btp, gm, bm,
           Wout, g1, b1, Wffn1, Wffn2, g2, b2):
    perm = jnp.asarray(_PERM)
    wq = Wq[:, perm]
    bqp = bq[perm].reshape(1, F)
    wk = Wk[:, perm]
    bkp = bk[perm].reshape(1, F)
    we = Wedge[:, perm]
    bep = bedge[perm].reshape(1, F)
    wtp0 = Wtp[:, 0, :]
    src = edge_index[0]
    dst = edge_index[1]
    zden = jnp.zeros((NPP, D), jnp.float32)
    zagg = jnp.zeros((NP, D), jnp.float32)

    qt, kt, p = _node_proj(node_features, wq, bqp, wk, bkp, Wv, wtp0)
    eft, shx = _edge_proj(edge_attr, we, bep, edge_sh)
    e, den = _pass_a(qt, kt, eft, dst, src, zden)
    denc = _den_comb(den)
    agg = _pass_b(e, denc, p, shx, dst, src, btp, gm, bm, zagg)
    return _post(node_features, agg, Wout,
                 g1.reshape(1, D), b1.reshape(1, D),
                 Wffn1, Wffn2, g2.reshape(1, D), b2.reshape(1, D))
